# baseline (speedup 1.0000x reference)
import functools

import jax
import jax.numpy as jnp
from jax import lax
from jax.experimental import pallas as pl
from jax.experimental.pallas import tpu as pltpu
from jax.experimental.pallas import tpu_sc as plsc

E, B, S, D = 8, 4, 2048, 1024
NC, NS = 2, 16
NW = NC * NS
WORKERS_PER_BATCH = NW // B
ROWS_PER_WORKER = S // WORKERS_PER_BATCH
CHUNK_ROWS = 8
NCHUNK = ROWS_PER_WORKER // CHUNK_ROWS
NBUF = 12
WOUT = 6

_mesh = plsc.VectorSubcoreMesh(core_axis_name="c", subcore_axis_name="s")


@functools.partial(
    pl.kernel,
    mesh=_mesh,
    out_type=jax.ShapeDtypeStruct((B, S, D), jnp.float32),
    scratch_types=[
        pltpu.VMEM((NBUF, CHUNK_ROWS, D), jnp.float32),
        pltpu.VMEM((32,), jnp.int32),
        pltpu.SemaphoreType.DMA,
        pltpu.SemaphoreType.DMA,
    ],
)
def _select_kernel(e0, e1, e2, e3, e4, e5, e6, e7, sel_hbm, out_hbm,
                   buf, sel_v, sem_in, sem_out):
    experts = (e0, e1, e2, e3, e4, e5, e6, e7)
    wid = lax.axis_index("s") * NC + lax.axis_index("c")
    b = wid // WORKERS_PER_BATCH
    row0 = (wid % WORKERS_PER_BATCH) * ROWS_PER_WORKER

    pltpu.sync_copy(sel_hbm, sel_v)
    sel_b = sel_v[pl.ds(b, 16)][0]

    for e in range(E):
        @pl.when(sel_b == e)
        def _(e=e):
            src = experts[e]

            def copy_in(c):
                return pltpu.async_copy(
                    src.at[b, pl.ds(row0 + c * CHUNK_ROWS, CHUNK_ROWS)],
                    buf.at[c % NBUF], sem_in)

            def copy_out(c):
                return pltpu.async_copy(
                    buf.at[c % NBUF],
                    out_hbm.at[b, pl.ds(row0 + c * CHUNK_ROWS, CHUNK_ROWS)],
                    sem_out)

            copies_in = [copy_in(c) for c in range(min(NBUF, NCHUNK))]
            copies_out = []
            next_in = len(copies_in)
            for c in range(NCHUNK):
                copies_in[c].wait()
                if c >= WOUT:
                    copies_out[c - WOUT].wait()
                    while next_in <= c - WOUT + NBUF and next_in < NCHUNK:
                        copies_in.append(copy_in(next_in))
                        next_in += 1
                copies_out.append(copy_out(c))
            for c in range(max(0, NCHUNK - WOUT), NCHUNK):
                copies_out[c].wait()


def kernel(expert_out_0, expert_out_1, expert_out_2, expert_out_3,
           expert_out_4, expert_out_5, expert_out_6, expert_out_7,
           selection_index):
    sel = jnp.zeros((32,), dtype=jnp.int32).at[:B].set(
        selection_index.astype(jnp.int32))
    return _select_kernel(
        expert_out_0, expert_out_1, expert_out_2, expert_out_3,
        expert_out_4, expert_out_5, expert_out_6, expert_out_7, sel)


# speedup vs baseline: 1.0133x; 1.0133x over previous
import functools

import jax
import jax.numpy as jnp
from jax import lax
from jax.experimental import pallas as pl
from jax.experimental.pallas import tpu as pltpu
from jax.experimental.pallas import tpu_sc as plsc

E, B, S, D = 8, 4, 2048, 1024
NC, NS = 2, 16
NW = NC * NS
WORKERS_PER_BATCH = NW // B
ROWS_PER_WORKER = S // WORKERS_PER_BATCH
CHUNK_ROWS = 16
NCHUNK = ROWS_PER_WORKER // CHUNK_ROWS
NBUF = 7
WOUT = 4

_mesh = plsc.VectorSubcoreMesh(core_axis_name="c", subcore_axis_name="s")


@functools.partial(
    pl.kernel,
    mesh=_mesh,
    out_type=jax.ShapeDtypeStruct((B, S, D), jnp.float32),
    scratch_types=[
        pltpu.VMEM((NBUF, CHUNK_ROWS, D), jnp.float32),
        pltpu.VMEM((32,), jnp.int32),
        pltpu.SemaphoreType.DMA,
        pltpu.SemaphoreType.DMA,
    ],
)
def _select_kernel(e0, e1, e2, e3, e4, e5, e6, e7, sel_hbm, out_hbm,
                   buf, sel_v, sem_in, sem_out):
    experts = (e0, e1, e2, e3, e4, e5, e6, e7)
    wid = lax.axis_index("s") * NC + lax.axis_index("c")
    b = wid // WORKERS_PER_BATCH
    row0 = (wid % WORKERS_PER_BATCH) * ROWS_PER_WORKER

    pltpu.sync_copy(sel_hbm, sel_v)
    sel_b = sel_v[pl.ds(b, 16)][0]

    for e in range(E):
        @pl.when(sel_b == e)
        def _(e=e):
            src = experts[e]

            def copy_in(c):
                return pltpu.async_copy(
                    src.at[b, pl.ds(row0 + c * CHUNK_ROWS, CHUNK_ROWS)],
                    buf.at[c % NBUF], sem_in)

            def copy_out(c):
                return pltpu.async_copy(
                    buf.at[c % NBUF],
                    out_hbm.at[b, pl.ds(row0 + c * CHUNK_ROWS, CHUNK_ROWS)],
                    sem_out)

            copies_in = [copy_in(c) for c in range(min(NBUF, NCHUNK))]
            copies_out = []
            next_in = len(copies_in)
            for c in range(NCHUNK):
                copies_in[c].wait()
                if c >= WOUT:
                    copies_out[c - WOUT].wait()
                    while next_in <= c - WOUT + NBUF and next_in < NCHUNK:
                        copies_in.append(copy_in(next_in))
                        next_in += 1
                copies_out.append(copy_out(c))
            for c in range(max(0, NCHUNK - WOUT), NCHUNK):
                copies_out[c].wait()


def kernel(expert_out_0, expert_out_1, expert_out_2, expert_out_3,
           expert_out_4, expert_out_5, expert_out_6, expert_out_7,
           selection_index):
    sel = jnp.zeros((32,), dtype=jnp.int32).at[:B].set(
        selection_index.astype(jnp.int32))
    return _select_kernel(
        expert_out_0, expert_out_1, expert_out_2, expert_out_3,
        expert_out_4, expert_out_5, expert_out_6, expert_out_7, sel)


# speedup vs baseline: 1.0148x; 1.0015x over previous
import functools

import jax
import jax.numpy as jnp
from jax import lax
from jax.experimental import pallas as pl
from jax.experimental.pallas import tpu as pltpu
from jax.experimental.pallas import tpu_sc as plsc

E, B, S, D = 8, 4, 2048, 1024
NC, NS = 2, 16
NW = NC * NS
WORKERS_PER_BATCH = NW // B
ROWS_PER_WORKER = S // WORKERS_PER_BATCH
CHUNK_ROWS = 16
NCHUNK = ROWS_PER_WORKER // CHUNK_ROWS
NBUF = 6
WOUT = 3

_mesh = plsc.VectorSubcoreMesh(core_axis_name="c", subcore_axis_name="s")


@functools.partial(
    pl.kernel,
    mesh=_mesh,
    out_type=jax.ShapeDtypeStruct((B, S, D), jnp.float32),
    scratch_types=[
        pltpu.VMEM((NBUF, CHUNK_ROWS, D), jnp.float32),
        pltpu.VMEM((32,), jnp.int32),
        pltpu.SemaphoreType.DMA,
        pltpu.SemaphoreType.DMA,
    ],
)
def _select_kernel(e0, e1, e2, e3, e4, e5, e6, e7, sel_hbm, out_hbm,
                   buf, sel_v, sem_in, sem_out):
    experts = (e0, e1, e2, e3, e4, e5, e6, e7)
    wid = lax.axis_index("s") * NC + lax.axis_index("c")
    b = wid // WORKERS_PER_BATCH
    row0 = (wid % WORKERS_PER_BATCH) * ROWS_PER_WORKER

    pltpu.sync_copy(sel_hbm, sel_v)
    sel_b = sel_v[pl.ds(b, 16)][0]

    for e in range(E):
        @pl.when(sel_b == e)
        def _(e=e):
            src = experts[e]

            def copy_in(c):
                return pltpu.async_copy(
                    src.at[b, pl.ds(row0 + c * CHUNK_ROWS, CHUNK_ROWS)],
                    buf.at[c % NBUF], sem_in)

            def copy_out(c):
                return pltpu.async_copy(
                    buf.at[c % NBUF],
                    out_hbm.at[b, pl.ds(row0 + c * CHUNK_ROWS, CHUNK_ROWS)],
                    sem_out)

            copies_in = [copy_in(c) for c in range(min(NBUF, NCHUNK))]
            copies_out = []
            next_in = len(copies_in)
            for c in range(NCHUNK):
                copies_in[c].wait()
                if c >= WOUT:
                    copies_out[c - WOUT].wait()
                    while next_in <= c - WOUT + NBUF and next_in < NCHUNK:
                        copies_in.append(copy_in(next_in))
                        next_in += 1
                copies_out.append(copy_out(c))
            for c in range(max(0, NCHUNK - WOUT), NCHUNK):
                copies_out[c].wait()


def kernel(expert_out_0, expert_out_1, expert_out_2, expert_out_3,
           expert_out_4, expert_out_5, expert_out_6, expert_out_7,
           selection_index):
    sel = jnp.zeros((32,), dtype=jnp.int32).at[:B].set(
        selection_index.astype(jnp.int32))
    return _select_kernel(
        expert_out_0, expert_out_1, expert_out_2, expert_out_3,
        expert_out_4, expert_out_5, expert_out_6, expert_out_7, sel)
